# Initial kernel scaffold; baseline (speedup 1.0000x reference)
import jax
import jax.numpy as jnp
from jax.experimental import pallas as pl


def kernel(token_ids, weight):
    raise NotImplementedError("write your pallas kernel here")


# speedup vs baseline: 1.0226x; 1.0226x over previous
import functools

import jax
import jax.numpy as jnp
from jax import lax
from jax.experimental import pallas as pl
from jax.experimental.pallas import tpu as pltpu
from jax.experimental.pallas import tpu_sc as plsc

NUM_TOKENS = 16384 * 50
DIM = 32
NC, NS = 2, 16
NW = NC * NS
PER_W = NUM_TOKENS // NW
CHUNK = 128
NCHUNK = PER_W // CHUNK

_mesh = plsc.VectorSubcoreMesh(core_axis_name="c", subcore_axis_name="s")


@functools.partial(
    pl.kernel,
    out_type=jax.ShapeDtypeStruct((NUM_TOKENS, DIM), jnp.float32),
    mesh=_mesh,
    scratch_types=[
        pltpu.VMEM((NCHUNK, CHUNK), jnp.int32),
        pltpu.VMEM((CHUNK, DIM), jnp.float32),
        pltpu.SemaphoreType.DMA,
    ],
    compiler_params=pltpu.CompilerParams(use_tc_tiling_on_sc=False),
)
def _embed_lookup(tok_hbm, table_hbm, out_hbm, idx_v, rows_v, sem):
    wid = lax.axis_index("s") * NC + lax.axis_index("c")
    base = wid * PER_W
    pltpu.sync_copy(tok_hbm.at[wid], idx_v)

    @pl.loop(0, NCHUNK)
    def _chunk(j):
        pltpu.async_copy(table_hbm.at[idx_v.at[j]], rows_v, sem).wait()
        pltpu.sync_copy(rows_v, out_hbm.at[pl.ds(base + j * CHUNK, CHUNK)])


def kernel(token_ids, weight):
    tok = token_ids.reshape(NW, NCHUNK, CHUNK).astype(jnp.int32)
    out = _embed_lookup(tok, weight)
    return out.reshape(token_ids.shape + (DIM,))


# speedup vs baseline: 1.1129x; 1.0882x over previous
import functools

import jax
import jax.numpy as jnp
from jax import lax
from jax.experimental import pallas as pl
from jax.experimental.pallas import tpu as pltpu
from jax.experimental.pallas import tpu_sc as plsc

NUM_TOKENS = 16384 * 50
DIM = 32
NC, NS = 2, 16
NW = NC * NS
PER_W = NUM_TOKENS // NW
CHUNK = 128
GROUP = 1024
SUB = GROUP // CHUNK
NGROUP = PER_W // GROUP
NCHUNK = PER_W // CHUNK

_mesh = plsc.VectorSubcoreMesh(core_axis_name="c", subcore_axis_name="s")


@functools.partial(
    pl.kernel,
    out_type=jax.ShapeDtypeStruct((NUM_TOKENS, DIM), jnp.float32),
    mesh=_mesh,
    scratch_types=[
        pltpu.VMEM((NCHUNK, CHUNK), jnp.int32),
        pltpu.VMEM((GROUP, DIM), jnp.float32),
        pltpu.VMEM((GROUP, DIM), jnp.float32),
        pltpu.SemaphoreType.DMA,
        pltpu.SemaphoreType.DMA,
        pltpu.SemaphoreType.DMA,
        pltpu.SemaphoreType.DMA,
    ],
    compiler_params=pltpu.CompilerParams(use_tc_tiling_on_sc=False),
)
def _embed_lookup(tok_hbm, table_hbm, out_hbm, idx_v, buf_a, buf_b,
                  sem_ga, sem_gb, sem_oa, sem_ob):
    wid = lax.axis_index("s") * NC + lax.axis_index("c")
    base = wid * PER_W
    pltpu.sync_copy(tok_hbm.at[wid], idx_v)

    def fire_gathers(g, buf, sem):
        for i in range(SUB):
            pltpu.async_copy(
                table_hbm.at[idx_v.at[g * SUB + i]],
                buf.at[pl.ds(i * CHUNK, CHUNK)],
                sem,
            )

    def drain_gathers(buf, sem):
        pltpu.make_async_copy(out_hbm.at[pl.ds(0, GROUP)], buf, sem).wait()

    def fire_copy(g, buf, sem):
        pltpu.async_copy(buf, out_hbm.at[pl.ds(base + g * GROUP, GROUP)], sem)

    def drain_copy(buf, sem):
        pltpu.make_async_copy(buf, out_hbm.at[pl.ds(0, GROUP)], sem).wait()

    fire_gathers(0, buf_a, sem_ga)
    fire_gathers(1, buf_b, sem_gb)
    drain_gathers(buf_a, sem_ga)
    fire_copy(0, buf_a, sem_oa)

    @pl.loop(0, (NGROUP - 3) // 2)
    def _pair(t):
        g = 2 * t + 1
        drain_copy(buf_a, sem_oa)
        fire_gathers(g + 1, buf_a, sem_ga)
        drain_gathers(buf_b, sem_gb)
        fire_copy(g, buf_b, sem_ob)
        drain_copy(buf_b, sem_ob)
        fire_gathers(g + 2, buf_b, sem_gb)
        drain_gathers(buf_a, sem_ga)
        fire_copy(g + 1, buf_a, sem_oa)

    g = NGROUP - 2
    drain_copy(buf_a, sem_oa)
    fire_gathers(g + 1, buf_a, sem_ga)
    drain_gathers(buf_b, sem_gb)
    fire_copy(g, buf_b, sem_ob)
    drain_copy(buf_b, sem_ob)
    drain_gathers(buf_a, sem_ga)
    fire_copy(g + 1, buf_a, sem_oa)
    drain_copy(buf_a, sem_oa)


def kernel(token_ids, weight):
    tok = token_ids.reshape(NW, NCHUNK, CHUNK).astype(jnp.int32)
    out = _embed_lookup(tok, weight)
    return out.reshape(token_ids.shape + (DIM,))


# speedup vs baseline: 1.2826x; 1.1525x over previous
import functools

import jax
import jax.numpy as jnp
from jax import lax
from jax.experimental import pallas as pl
from jax.experimental.pallas import tpu as pltpu
from jax.experimental.pallas import tpu_sc as plsc

NUM_TOKENS = 16384 * 50
DIM = 32
NC, NS = 2, 16
NW = NC * NS
PER_W = NUM_TOKENS // NW
CHUNK = 128
GROUP = 1024
SUB = GROUP // CHUNK
NGROUP = PER_W // GROUP
NCHUNK = PER_W // CHUNK

_mesh = plsc.VectorSubcoreMesh(core_axis_name="c", subcore_axis_name="s")


@functools.partial(
    pl.kernel,
    out_type=jax.ShapeDtypeStruct((NW * NGROUP, GROUP, DIM), jnp.float32),
    mesh=_mesh,
    scratch_types=[
        pltpu.VMEM((NGROUP, GROUP), jnp.int32),
        pltpu.VMEM((GROUP, DIM), jnp.float32),
        pltpu.VMEM((GROUP, DIM), jnp.float32),
        pltpu.SemaphoreType.DMA,
        pltpu.SemaphoreType.DMA,
        pltpu.SemaphoreType.DMA,
        pltpu.SemaphoreType.DMA,
    ],
    compiler_params=pltpu.CompilerParams(use_tc_tiling_on_sc=False),
)
def _embed_lookup(tok_hbm, table_hbm, out_hbm, idx_v, buf_a, buf_b,
                  sem_ga, sem_gb, sem_oa, sem_ob):
    wid = lax.axis_index("s") * NC + lax.axis_index("c")
    gbase = wid * NGROUP
    pltpu.sync_copy(tok_hbm.at[wid], idx_v)

    def fire_gather(g, buf, sem):
        pltpu.async_copy(table_hbm.at[idx_v.at[g]], buf, sem)

    def drain_gather(buf, sem):
        pltpu.make_async_copy(out_hbm.at[0], buf, sem).wait()

    def fire_copy(g, buf, sem):
        pltpu.async_copy(buf, out_hbm.at[gbase + g], sem)

    def drain_copy(buf, sem):
        pltpu.make_async_copy(buf, out_hbm.at[0], sem).wait()

    fire_gather(0, buf_a, sem_ga)
    fire_gather(1, buf_b, sem_gb)
    drain_gather(buf_a, sem_ga)
    fire_copy(0, buf_a, sem_oa)

    @pl.loop(0, (NGROUP - 3) // 2)
    def _pair(t):
        g = 2 * t + 1
        drain_copy(buf_a, sem_oa)
        fire_gather(g + 1, buf_a, sem_ga)
        drain_gather(buf_b, sem_gb)
        fire_copy(g, buf_b, sem_ob)
        drain_copy(buf_b, sem_ob)
        fire_gather(g + 2, buf_b, sem_gb)
        drain_gather(buf_a, sem_ga)
        fire_copy(g + 1, buf_a, sem_oa)

    g = NGROUP - 2
    drain_copy(buf_a, sem_oa)
    fire_gather(g + 1, buf_a, sem_ga)
    drain_gather(buf_b, sem_gb)
    fire_copy(g, buf_b, sem_ob)
    drain_copy(buf_b, sem_ob)
    drain_gather(buf_a, sem_ga)
    fire_copy(g + 1, buf_a, sem_oa)
    drain_copy(buf_a, sem_oa)


def kernel(token_ids, weight):
    tok = token_ids.reshape(NW, NGROUP, GROUP).astype(jnp.int32)
    out = _embed_lookup(tok, weight)
    return out.reshape(token_ids.shape + (DIM,))
